# baseline (speedup 1.0000x reference)
import jax
import jax.numpy as jnp
from jax import lax
from jax.experimental import pallas as pl
from jax.experimental.pallas import tpu as pltpu
from jax.experimental.pallas import tpu_sc as plsc

N_U = 100000
N_I = 100000
K = 128
D = 256
F = 1024
B = 16384

NW = 32
RPW = B // NW
C_VF = 8
N_VF = RPW // C_VF
C_L = 32
N_L = RPW // C_L
DP = 384


def _sc_body(u_hbm, i_hbm, j_hbm, vf_hbm, ulat_hbm, ilat_hbm, uvis_hbm,
             ibias_hbm, dvf_out, uvis_out, latp_out, bdiff_out,
             idx_u, idx_i, idx_j,
             va0, va1, vb0, vb1, vo0, vo1,
             ul0, ul1, ul2, li0, li1, li2, lj0, lj1, lj2,
             uv0, uv1, uv2, lp0, lp1, lp2,
             bias_i_v, bias_j_v, bdiff_v,
             sidx, vg0, vg1, vw0, vw1, lg0, lg1, lg2, lw0, lw1, lw2):
    wid = lax.axis_index("s") * 2 + lax.axis_index("c")
    base = wid * RPW

    pltpu.sync_copy(u_hbm.at[pl.ds(base, RPW)], idx_u)
    pltpu.sync_copy(i_hbm.at[pl.ds(base, RPW)], idx_i)
    pltpu.sync_copy(j_hbm.at[pl.ds(base, RPW)], idx_j)

    va = (va0, va1)
    vb = (vb0, vb1)
    vo = (vo0, vo1)
    vg = (vg0, vg1)
    vw = (vw0, vw1)

    def vf_gather(c, slot):
        off = c * C_VF
        pltpu.async_copy(vf_hbm.at[idx_i.at[pl.ds(off, C_VF)]], va[slot],
                         vg[slot])
        pltpu.async_copy(vf_hbm.at[idx_j.at[pl.ds(off, C_VF)]], vb[slot],
                         vg[slot])

    def vf_gather_wait(c, slot):
        off = c * C_VF
        pltpu.make_async_copy(vf_hbm.at[idx_i.at[pl.ds(off, C_VF)]], va[slot],
                              vg[slot]).wait()
        pltpu.make_async_copy(vf_hbm.at[idx_j.at[pl.ds(off, C_VF)]], vb[slot],
                              vg[slot]).wait()

    def vf_write(c, slot):
        off = pl.multiple_of((base + c * C_VF) * F, C_VF * F)
        pltpu.async_copy(
            vo[slot], dvf_out.at[pl.ds(off, C_VF * F)], vw[slot])

    def vf_write_wait(c, slot):
        off = pl.multiple_of((base + c * C_VF) * F, C_VF * F)
        pltpu.make_async_copy(
            vo[slot], dvf_out.at[pl.ds(off, C_VF * F)], vw[slot]).wait()

    vf_gather(0, 0)
    vf_gather(1, 1)

    cpi = pltpu.async_copy(ibias_hbm.at[idx_i], bias_i_v, sidx)
    cpj = pltpu.async_copy(ibias_hbm.at[idx_j], bias_j_v, sidx)
    cpi.wait()
    cpj.wait()
    for k in range(RPW // 16):
        s = pl.ds(k * 16, 16)
        bdiff_v[s] = bias_i_v[s] - bias_j_v[s]
    pltpu.sync_copy(bdiff_v, bdiff_out.at[pl.ds(base, RPW)])

    def vf_visit(c, slot):
        @pl.when(c >= 2)
        def _():
            vf_write_wait(c - 2, slot)
        vf_gather_wait(c, slot)

        def sub_row(r, _):
            row_off = pl.multiple_of(r * F, F)
            for k in range(F // 16):
                s = pl.ds(k * 16, 16)
                vo[slot][pl.ds(row_off + k * 16, 16)] = (
                    va[slot][r, s] - vb[slot][r, s])
            return 0
        lax.fori_loop(0, C_VF, sub_row, 0)

        @pl.when(c + 2 < N_VF)
        def _():
            vf_gather(c + 2, slot)
        vf_write(c, slot)

    def vf_iter(c2, _):
        c = c2 * 2
        vf_visit(c, 0)
        vf_visit(c + 1, 1)
        return 0
    lax.fori_loop(0, N_VF // 2, vf_iter, 0)
    vf_write_wait(N_VF - 2, 0)
    vf_write_wait(N_VF - 1, 1)

    ul = (ul0, ul1, ul2)
    li = (li0, li1, li2)
    lj = (lj0, lj1, lj2)
    uv = (uv0, uv1, uv2)
    lp = (lp0, lp1, lp2)
    lg = (lg0, lg1, lg2)
    lw = (lw0, lw1, lw2)

    def l_gather(c, slot):
        off = c * C_L
        iu = idx_u.at[pl.ds(off, C_L)]
        ii = idx_i.at[pl.ds(off, C_L)]
        ij = idx_j.at[pl.ds(off, C_L)]
        pltpu.async_copy(ulat_hbm.at[iu], ul[slot], lg[slot])
        pltpu.async_copy(ilat_hbm.at[ii], li[slot], lg[slot])
        pltpu.async_copy(ilat_hbm.at[ij], lj[slot], lg[slot])
        pltpu.async_copy(uvis_hbm.at[iu], uv[slot], lg[slot])

    def l_gather_wait(c, slot):
        off = c * C_L
        iu = idx_u.at[pl.ds(off, C_L)]
        ii = idx_i.at[pl.ds(off, C_L)]
        ij = idx_j.at[pl.ds(off, C_L)]
        pltpu.make_async_copy(ulat_hbm.at[iu], ul[slot], lg[slot]).wait()
        pltpu.make_async_copy(ilat_hbm.at[ii], li[slot], lg[slot]).wait()
        pltpu.make_async_copy(ilat_hbm.at[ij], lj[slot], lg[slot]).wait()
        pltpu.make_async_copy(uvis_hbm.at[iu], uv[slot], lg[slot]).wait()

    def l_write(c, slot):
        off = base + c * C_L
        pltpu.async_copy(lp[slot], latp_out.at[pl.ds(off, C_L)], lw[slot])
        pltpu.async_copy(uv[slot], uvis_out.at[pl.ds(off, C_L)], lw[slot])

    def l_write_wait(c, slot):
        off = base + c * C_L
        pltpu.make_async_copy(lp[slot], latp_out.at[pl.ds(off, C_L)],
                              lw[slot]).wait()
        pltpu.make_async_copy(uv[slot], uvis_out.at[pl.ds(off, C_L)],
                              lw[slot]).wait()

    l_gather(0, 0)
    l_gather(1, 1)
    for c in range(N_L):
        slot = c % 3
        l_gather_wait(c, slot)

        def dot_row(r, _):
            acc = jnp.zeros((16,), jnp.float32)
            for k in range(K // 16):
                s = pl.ds(k * 16, 16)
                acc = acc + ul[slot][r, s] * (li[slot][r, s] - lj[slot][r, s])
            lp[slot][r, pl.ds(0, 16)] = acc
            return 0
        lax.fori_loop(0, C_L, dot_row, 0)

        l_write(c, slot)
        if c + 2 < N_L:
            nslot = (c + 2) % 3
            if c - 1 >= 0:
                l_write_wait(c - 1, nslot)
            l_gather(c + 2, nslot)
    for c in range(N_L - 3, N_L):
        l_write_wait(c, c % 3)


def _sc_gather(u, i, j, vf, ulat, ilat, uvisf, ibias):
    mesh = plsc.VectorSubcoreMesh(core_axis_name="c", subcore_axis_name="s")
    f32 = jnp.float32
    return pl.kernel(
        _sc_body,
        out_type=[
            jax.ShapeDtypeStruct((B * F,), f32),
            jax.ShapeDtypeStruct((B, D), f32),
            jax.ShapeDtypeStruct((B, 16), f32),
            jax.ShapeDtypeStruct((B,), f32),
        ],
        mesh=mesh,
        scratch_types=[
            pltpu.VMEM((RPW,), jnp.int32),
            pltpu.VMEM((RPW,), jnp.int32),
            pltpu.VMEM((RPW,), jnp.int32),
            pltpu.VMEM((C_VF, F), f32),
            pltpu.VMEM((C_VF, F), f32),
            pltpu.VMEM((C_VF, F), f32),
            pltpu.VMEM((C_VF, F), f32),
            pltpu.VMEM((C_VF * F,), f32),
            pltpu.VMEM((C_VF * F,), f32),
            pltpu.VMEM((C_L, K), f32),
            pltpu.VMEM((C_L, K), f32),
            pltpu.VMEM((C_L, K), f32),
            pltpu.VMEM((C_L, K), f32),
            pltpu.VMEM((C_L, K), f32),
            pltpu.VMEM((C_L, K), f32),
            pltpu.VMEM((C_L, K), f32),
            pltpu.VMEM((C_L, K), f32),
            pltpu.VMEM((C_L, K), f32),
            pltpu.VMEM((C_L, D), f32),
            pltpu.VMEM((C_L, D), f32),
            pltpu.VMEM((C_L, D), f32),
            pltpu.VMEM((C_L, 16), f32),
            pltpu.VMEM((C_L, 16), f32),
            pltpu.VMEM((C_L, 16), f32),
            pltpu.VMEM((RPW,), f32),
            pltpu.VMEM((RPW,), f32),
            pltpu.VMEM((RPW,), f32),
            pltpu.SemaphoreType.DMA,
            pltpu.SemaphoreType.DMA,
            pltpu.SemaphoreType.DMA,
            pltpu.SemaphoreType.DMA,
            pltpu.SemaphoreType.DMA,
            pltpu.SemaphoreType.DMA,
            pltpu.SemaphoreType.DMA,
            pltpu.SemaphoreType.DMA,
            pltpu.SemaphoreType.DMA,
            pltpu.SemaphoreType.DMA,
            pltpu.SemaphoreType.DMA,
        ],
    )(u, i, j, vf, ulat, ilat, uvisf, ibias)


BM = 1024


def _tc_body(dvf_ref, wt_ref, uvis_ref, latp_ref, bdiff_ref, out_ref):
    p = jnp.dot(dvf_ref[...], wt_ref[...], preferred_element_type=jnp.float32)
    vis = jnp.sum(p[:, :D] * uvis_ref[...], axis=1)
    beta = jnp.sum(p[:, D:], axis=1)
    lat = jnp.sum(latp_ref[...], axis=1)
    out_ref[0, 0, :] = vis + beta + lat + bdiff_ref[0, 0, :]


def _tc_combine(dvf, wt, uvis, latp, bdiff2d):
    grid = (B // BM,)
    return pl.pallas_call(
        _tc_body,
        grid=grid,
        in_specs=[
            pl.BlockSpec((BM, F), lambda m: (m, 0)),
            pl.BlockSpec((F, DP), lambda m: (0, 0)),
            pl.BlockSpec((BM, D), lambda m: (m, 0)),
            pl.BlockSpec((BM, 16), lambda m: (m, 0)),
            pl.BlockSpec((1, 1, BM), lambda m: (m, 0, 0)),
        ],
        out_specs=pl.BlockSpec((1, 1, BM), lambda m: (m, 0, 0)),
        out_shape=jax.ShapeDtypeStruct((B // BM, 1, BM), jnp.float32),
    )(dvf, wt, uvis, latp, bdiff2d)


@jax.jit
def kernel(trg_batch, U_latent_factors, I_latent_factors, U_visual_factors,
           W_proj, b_proj, beta_dash, user_bias, item_bias, visual_features):
    u = trg_batch[:, 0].astype(jnp.int32)
    i = trg_batch[:, 1].astype(jnp.int32)
    j = trg_batch[:, 2].astype(jnp.int32)

    dvf_flat, uvis, latp, bdiff = _sc_gather(
        u, i, j, visual_features, U_latent_factors, I_latent_factors,
        U_visual_factors, item_bias)

    wt = jnp.concatenate(
        [W_proj, beta_dash, jnp.zeros((DP - D - 1, F), jnp.float32)], axis=0).T

    out3d = _tc_combine(dvf_flat.reshape(B, F), wt, uvis, latp,
                        bdiff.reshape(B // BM, 1, BM))
    return out3d.reshape(B)


# speedup vs baseline: 1.9004x; 1.9004x over previous
import jax
import jax.numpy as jnp
from jax import lax
from jax.experimental import pallas as pl
from jax.experimental.pallas import tpu as pltpu
from jax.experimental.pallas import tpu_sc as plsc

N_U = 100000
N_I = 100000
K = 128
D = 256
F = 1024
B = 16384

NW = 32
RPW = B // NW
C_VF = 8
N_VF = RPW // C_VF
C_L = 32
N_L = RPW // C_L
DP = 384


def _sc_body(u_hbm, i_hbm, j_hbm, vf_hbm, ulat_hbm, ilat_hbm, uvis_hbm,
             ibias_hbm, dvf_out, uvis_out, latp_out, bdiff_out,
             idx_u, idx_i, idx_j,
             va0, va1, vb0, vb1, vo0, vo1,
             ul0, ul1, ul2, li0, li1, li2, lj0, lj1, lj2,
             uv0, uv1, uv2, lp0, lp1, lp2,
             bias_i_v, bias_j_v, bdiff_v,
             sidx, vg0, vg1, vw0, vw1, lg0, lg1, lg2, lw0, lw1, lw2):
    wid = lax.axis_index("s") * 2 + lax.axis_index("c")
    base = wid * RPW

    pltpu.sync_copy(u_hbm.at[pl.ds(base, RPW)], idx_u)
    pltpu.sync_copy(i_hbm.at[pl.ds(base, RPW)], idx_i)
    pltpu.sync_copy(j_hbm.at[pl.ds(base, RPW)], idx_j)

    va = (va0, va1)
    vb = (vb0, vb1)
    vo = (vo0, vo1)
    vg = (vg0, vg1)
    vw = (vw0, vw1)

    def vf_gather(c, slot):
        off = c * C_VF
        pltpu.async_copy(vf_hbm.at[idx_i.at[pl.ds(off, C_VF)]], va[slot],
                         vg[slot])
        pltpu.async_copy(vf_hbm.at[idx_j.at[pl.ds(off, C_VF)]], vb[slot],
                         vg[slot])

    def vf_gather_wait(c, slot):
        off = c * C_VF
        pltpu.make_async_copy(vf_hbm.at[idx_i.at[pl.ds(off, C_VF)]], va[slot],
                              vg[slot]).wait()
        pltpu.make_async_copy(vf_hbm.at[idx_j.at[pl.ds(off, C_VF)]], vb[slot],
                              vg[slot]).wait()

    def vf_write(c, slot):
        pltpu.async_copy(
            vo[slot], dvf_out.at[pl.ds(base + c * C_VF, C_VF)], vw[slot])

    def vf_write_wait(c, slot):
        pltpu.make_async_copy(
            vo[slot], dvf_out.at[pl.ds(base + c * C_VF, C_VF)],
            vw[slot]).wait()

    vf_gather(0, 0)
    vf_gather(1, 1)

    cpi = pltpu.async_copy(ibias_hbm.at[idx_i], bias_i_v, sidx)
    cpj = pltpu.async_copy(ibias_hbm.at[idx_j], bias_j_v, sidx)
    cpi.wait()
    cpj.wait()
    for k in range(RPW // 16):
        s = pl.ds(k * 16, 16)
        bdiff_v[s] = bias_i_v[s] - bias_j_v[s]
    pltpu.sync_copy(bdiff_v, bdiff_out.at[pl.ds(base, RPW)])

    def vf_visit(c, slot):
        @pl.when(c >= 2)
        def _():
            vf_write_wait(c - 2, slot)
        vf_gather_wait(c, slot)

        def sub_row(r, _):
            for k in range(F // 16):
                s = pl.ds(k * 16, 16)
                vo[slot][r, s] = va[slot][r, s] - vb[slot][r, s]
            return 0
        lax.fori_loop(0, C_VF, sub_row, 0)

        @pl.when(c + 2 < N_VF)
        def _():
            vf_gather(c + 2, slot)
        vf_write(c, slot)

    def vf_iter(c2, _):
        c = c2 * 2
        vf_visit(c, 0)
        vf_visit(c + 1, 1)
        return 0
    lax.fori_loop(0, N_VF // 2, vf_iter, 0)
    vf_write_wait(N_VF - 2, 0)
    vf_write_wait(N_VF - 1, 1)

    ul = (ul0, ul1, ul2)
    li = (li0, li1, li2)
    lj = (lj0, lj1, lj2)
    uv = (uv0, uv1, uv2)
    lp = (lp0, lp1, lp2)
    lg = (lg0, lg1, lg2)
    lw = (lw0, lw1, lw2)

    def l_gather(c, slot):
        off = c * C_L
        iu = idx_u.at[pl.ds(off, C_L)]
        ii = idx_i.at[pl.ds(off, C_L)]
        ij = idx_j.at[pl.ds(off, C_L)]
        pltpu.async_copy(ulat_hbm.at[iu], ul[slot], lg[slot])
        pltpu.async_copy(ilat_hbm.at[ii], li[slot], lg[slot])
        pltpu.async_copy(ilat_hbm.at[ij], lj[slot], lg[slot])
        pltpu.async_copy(uvis_hbm.at[iu], uv[slot], lg[slot])

    def l_gather_wait(c, slot):
        off = c * C_L
        iu = idx_u.at[pl.ds(off, C_L)]
        ii = idx_i.at[pl.ds(off, C_L)]
        ij = idx_j.at[pl.ds(off, C_L)]
        pltpu.make_async_copy(ulat_hbm.at[iu], ul[slot], lg[slot]).wait()
        pltpu.make_async_copy(ilat_hbm.at[ii], li[slot], lg[slot]).wait()
        pltpu.make_async_copy(ilat_hbm.at[ij], lj[slot], lg[slot]).wait()
        pltpu.make_async_copy(uvis_hbm.at[iu], uv[slot], lg[slot]).wait()

    def l_write(c, slot):
        off = base + c * C_L
        pltpu.async_copy(lp[slot], latp_out.at[pl.ds(off, C_L)], lw[slot])
        pltpu.async_copy(uv[slot], uvis_out.at[pl.ds(off, C_L)], lw[slot])

    def l_write_wait(c, slot):
        off = base + c * C_L
        pltpu.make_async_copy(lp[slot], latp_out.at[pl.ds(off, C_L)],
                              lw[slot]).wait()
        pltpu.make_async_copy(uv[slot], uvis_out.at[pl.ds(off, C_L)],
                              lw[slot]).wait()

    l_gather(0, 0)
    l_gather(1, 1)
    for c in range(N_L):
        slot = c % 3
        l_gather_wait(c, slot)

        def dot_row(r, _):
            acc = jnp.zeros((16,), jnp.float32)
            for k in range(K // 16):
                s = pl.ds(k * 16, 16)
                acc = acc + ul[slot][r, s] * (li[slot][r, s] - lj[slot][r, s])
            lp[slot][r, pl.ds(0, 16)] = acc
            return 0
        lax.fori_loop(0, C_L, dot_row, 0)

        l_write(c, slot)
        if c + 2 < N_L:
            nslot = (c + 2) % 3
            if c - 1 >= 0:
                l_write_wait(c - 1, nslot)
            l_gather(c + 2, nslot)
    for c in range(N_L - 3, N_L):
        l_write_wait(c, c % 3)


def _sc_gather(u, i, j, vf, ulat, ilat, uvisf, ibias):
    mesh = plsc.VectorSubcoreMesh(core_axis_name="c", subcore_axis_name="s")
    f32 = jnp.float32
    return pl.kernel(
        _sc_body,
        out_type=[
            jax.ShapeDtypeStruct((B, F), f32),
            jax.ShapeDtypeStruct((B, D), f32),
            jax.ShapeDtypeStruct((B, 16), f32),
            jax.ShapeDtypeStruct((B,), f32),
        ],
        mesh=mesh,
        scratch_types=[
            pltpu.VMEM((RPW,), jnp.int32),
            pltpu.VMEM((RPW,), jnp.int32),
            pltpu.VMEM((RPW,), jnp.int32),
            pltpu.VMEM((C_VF, F), f32),
            pltpu.VMEM((C_VF, F), f32),
            pltpu.VMEM((C_VF, F), f32),
            pltpu.VMEM((C_VF, F), f32),
            pltpu.VMEM((C_VF, F), f32),
            pltpu.VMEM((C_VF, F), f32),
            pltpu.VMEM((C_L, K), f32),
            pltpu.VMEM((C_L, K), f32),
            pltpu.VMEM((C_L, K), f32),
            pltpu.VMEM((C_L, K), f32),
            pltpu.VMEM((C_L, K), f32),
            pltpu.VMEM((C_L, K), f32),
            pltpu.VMEM((C_L, K), f32),
            pltpu.VMEM((C_L, K), f32),
            pltpu.VMEM((C_L, K), f32),
            pltpu.VMEM((C_L, D), f32),
            pltpu.VMEM((C_L, D), f32),
            pltpu.VMEM((C_L, D), f32),
            pltpu.VMEM((C_L, 16), f32),
            pltpu.VMEM((C_L, 16), f32),
            pltpu.VMEM((C_L, 16), f32),
            pltpu.VMEM((RPW,), f32),
            pltpu.VMEM((RPW,), f32),
            pltpu.VMEM((RPW,), f32),
            pltpu.SemaphoreType.DMA,
            pltpu.SemaphoreType.DMA,
            pltpu.SemaphoreType.DMA,
            pltpu.SemaphoreType.DMA,
            pltpu.SemaphoreType.DMA,
            pltpu.SemaphoreType.DMA,
            pltpu.SemaphoreType.DMA,
            pltpu.SemaphoreType.DMA,
            pltpu.SemaphoreType.DMA,
            pltpu.SemaphoreType.DMA,
            pltpu.SemaphoreType.DMA,
        ],
    )(u, i, j, vf, ulat, ilat, uvisf, ibias)


BM = 1024


def _tc_body(dvf_ref, wt_ref, uvis_ref, latp_ref, bdiff_ref, out_ref):
    p = jnp.dot(dvf_ref[...], wt_ref[...], preferred_element_type=jnp.float32)
    vis = jnp.sum(p[:, :D] * uvis_ref[...], axis=1)
    beta = jnp.sum(p[:, D:], axis=1)
    lat = jnp.sum(latp_ref[...], axis=1)
    out_ref[0, 0, :] = vis + beta + lat + bdiff_ref[0, 0, :]


def _tc_combine(dvf, wt, uvis, latp, bdiff2d):
    grid = (B // BM,)
    return pl.pallas_call(
        _tc_body,
        grid=grid,
        in_specs=[
            pl.BlockSpec((BM, F), lambda m: (m, 0)),
            pl.BlockSpec((F, DP), lambda m: (0, 0)),
            pl.BlockSpec((BM, D), lambda m: (m, 0)),
            pl.BlockSpec((BM, 16), lambda m: (m, 0)),
            pl.BlockSpec((1, 1, BM), lambda m: (m, 0, 0)),
        ],
        out_specs=pl.BlockSpec((1, 1, BM), lambda m: (m, 0, 0)),
        out_shape=jax.ShapeDtypeStruct((B // BM, 1, BM), jnp.float32),
    )(dvf, wt, uvis, latp, bdiff2d)


@jax.jit
def kernel(trg_batch, U_latent_factors, I_latent_factors, U_visual_factors,
           W_proj, b_proj, beta_dash, user_bias, item_bias, visual_features):
    u = trg_batch[:, 0].astype(jnp.int32)
    i = trg_batch[:, 1].astype(jnp.int32)
    j = trg_batch[:, 2].astype(jnp.int32)

    dvf, uvis, latp, bdiff = _sc_gather(
        u, i, j, visual_features, U_latent_factors, I_latent_factors,
        U_visual_factors, item_bias)

    wt = jnp.concatenate(
        [W_proj, beta_dash, jnp.zeros((DP - D - 1, F), jnp.float32)], axis=0).T

    out3d = _tc_combine(dvf, wt, uvis, latp,
                        bdiff.reshape(B // BM, 1, BM))
    return out3d.reshape(B)


# speedup vs baseline: 1.9087x; 1.0043x over previous
import jax
import jax.numpy as jnp
from jax import lax
from jax.experimental import pallas as pl
from jax.experimental.pallas import tpu as pltpu
from jax.experimental.pallas import tpu_sc as plsc

N_U = 100000
N_I = 100000
K = 128
D = 256
F = 1024
B = 16384

NW = 32
RPW = B // NW
C_VF = 8
N_VF = RPW // C_VF
C_L = 32
N_L = RPW // C_L
DP = 384


def _sc_body(u_hbm, i_hbm, j_hbm, vf_hbm, ulat_hbm, ilat_hbm, uvis_hbm,
             ibias_hbm, dvf_out, uvis_out, latp_out, bdiff_out,
             idx_u, idx_i, idx_j,
             va0, va1, vb0, vb1, vo0, vo1,
             ul0, ul1, ul2, li0, li1, li2, lj0, lj1, lj2,
             uv0, uv1, uv2, lp0, lp1, lp2,
             bias_i_v, bias_j_v, bdiff_v,
             sidx, vg0, vg1, vw0, vw1, lg0, lg1, lg2, lw0, lw1, lw2):
    wid = lax.axis_index("s") * 2 + lax.axis_index("c")
    base = wid * RPW

    pltpu.sync_copy(u_hbm.at[pl.ds(base, RPW)], idx_u)
    pltpu.sync_copy(i_hbm.at[pl.ds(base, RPW)], idx_i)
    pltpu.sync_copy(j_hbm.at[pl.ds(base, RPW)], idx_j)

    va = (va0, va1)
    vb = (vb0, vb1)
    vo = (vo0, vo1)
    vg = (vg0, vg1)
    vw = (vw0, vw1)

    def vf_gather(c, slot):
        off = c * C_VF
        pltpu.async_copy(vf_hbm.at[idx_i.at[pl.ds(off, C_VF)]], va[slot],
                         vg[slot])
        pltpu.async_copy(vf_hbm.at[idx_j.at[pl.ds(off, C_VF)]], vb[slot],
                         vg[slot])

    def vf_gather_wait(c, slot):
        off = c * C_VF
        pltpu.make_async_copy(vf_hbm.at[idx_i.at[pl.ds(off, C_VF)]], va[slot],
                              vg[slot]).wait()
        pltpu.make_async_copy(vf_hbm.at[idx_j.at[pl.ds(off, C_VF)]], vb[slot],
                              vg[slot]).wait()

    def vf_write(c, slot):
        pltpu.async_copy(
            vo[slot], dvf_out.at[pl.ds(base + c * C_VF, C_VF)], vw[slot])

    def vf_write_wait(c, slot):
        pltpu.make_async_copy(
            vo[slot], dvf_out.at[pl.ds(base + c * C_VF, C_VF)],
            vw[slot]).wait()

    vf_gather(0, 0)
    vf_gather(1, 1)

    cpi = pltpu.async_copy(ibias_hbm.at[idx_i], bias_i_v, sidx)
    cpj = pltpu.async_copy(ibias_hbm.at[idx_j], bias_j_v, sidx)
    cpi.wait()
    cpj.wait()
    for k in range(RPW // 16):
        s = pl.ds(k * 16, 16)
        bdiff_v[s] = bias_i_v[s] - bias_j_v[s]
    pltpu.sync_copy(bdiff_v, bdiff_out.at[pl.ds(base, RPW)])

    def vf_visit(c, slot):
        @pl.when(c >= 2)
        def _():
            vf_write_wait(c - 2, slot)
        vf_gather_wait(c, slot)

        def sub_row(r, _):
            for k in range(F // 16):
                s = pl.ds(k * 16, 16)
                vo[slot][r, s] = va[slot][r, s] - vb[slot][r, s]
            return 0
        lax.fori_loop(0, C_VF, sub_row, 0)

        @pl.when(c + 2 < N_VF)
        def _():
            vf_gather(c + 2, slot)
        vf_write(c, slot)

    def vf_iter(c2, _):
        c = c2 * 2
        vf_visit(c, 0)
        vf_visit(c + 1, 1)
        return 0
    lax.fori_loop(0, N_VF // 2, vf_iter, 0)
    vf_write_wait(N_VF - 2, 0)
    vf_write_wait(N_VF - 1, 1)

    ul = (ul0, ul1, ul2)
    li = (li0, li1, li2)
    lj = (lj0, lj1, lj2)
    uv = (uv0, uv1, uv2)
    lp = (lp0, lp1, lp2)
    lg = (lg0, lg1, lg2)
    lw = (lw0, lw1, lw2)

    def l_gather(c, slot):
        off = c * C_L
        iu = idx_u.at[pl.ds(off, C_L)]
        ii = idx_i.at[pl.ds(off, C_L)]
        ij = idx_j.at[pl.ds(off, C_L)]
        pltpu.async_copy(ulat_hbm.at[iu], ul[slot], lg[slot])
        pltpu.async_copy(ilat_hbm.at[ii], li[slot], lg[slot])
        pltpu.async_copy(ilat_hbm.at[ij], lj[slot], lg[slot])
        pltpu.async_copy(uvis_hbm.at[iu], uv[slot], lg[slot])

    def l_gather_wait(c, slot):
        off = c * C_L
        iu = idx_u.at[pl.ds(off, C_L)]
        ii = idx_i.at[pl.ds(off, C_L)]
        ij = idx_j.at[pl.ds(off, C_L)]
        pltpu.make_async_copy(ulat_hbm.at[iu], ul[slot], lg[slot]).wait()
        pltpu.make_async_copy(ilat_hbm.at[ii], li[slot], lg[slot]).wait()
        pltpu.make_async_copy(ilat_hbm.at[ij], lj[slot], lg[slot]).wait()
        pltpu.make_async_copy(uvis_hbm.at[iu], uv[slot], lg[slot]).wait()

    def l_write(c, slot):
        off = base + c * C_L
        pltpu.async_copy(lp[slot], latp_out.at[pl.ds(off, C_L)], lw[slot])
        pltpu.async_copy(uv[slot], uvis_out.at[pl.ds(off, C_L)], lw[slot])

    def l_write_wait(c, slot):
        off = base + c * C_L
        pltpu.make_async_copy(lp[slot], latp_out.at[pl.ds(off, C_L)],
                              lw[slot]).wait()
        pltpu.make_async_copy(uv[slot], uvis_out.at[pl.ds(off, C_L)],
                              lw[slot]).wait()

    l_gather(0, 0)
    l_gather(1, 1)
    for c in range(N_L):
        slot = c % 3
        l_gather_wait(c, slot)

        def dot_row(r, _):
            acc = jnp.zeros((16,), jnp.float32)
            for k in range(K // 16):
                s = pl.ds(k * 16, 16)
                acc = acc + ul[slot][r, s] * (li[slot][r, s] - lj[slot][r, s])
            lp[slot][r, pl.ds(0, 16)] = acc
            return 0
        lax.fori_loop(0, C_L, dot_row, 0)

        l_write(c, slot)
        if c + 2 < N_L:
            nslot = (c + 2) % 3
            if c - 1 >= 0:
                l_write_wait(c - 1, nslot)
            l_gather(c + 2, nslot)
    for c in range(N_L - 3, N_L):
        l_write_wait(c, c % 3)


def _sc_gather(u, i, j, vf, ulat, ilat, uvisf, ibias):
    mesh = plsc.VectorSubcoreMesh(core_axis_name="c", subcore_axis_name="s")
    f32 = jnp.float32
    return pl.kernel(
        _sc_body,
        out_type=[
            jax.ShapeDtypeStruct((B, F), f32),
            jax.ShapeDtypeStruct((B, D), f32),
            jax.ShapeDtypeStruct((B, 16), f32),
            jax.ShapeDtypeStruct((B,), f32),
        ],
        mesh=mesh,
        scratch_types=[
            pltpu.VMEM((RPW,), jnp.int32),
            pltpu.VMEM((RPW,), jnp.int32),
            pltpu.VMEM((RPW,), jnp.int32),
            pltpu.VMEM((C_VF, F), f32),
            pltpu.VMEM((C_VF, F), f32),
            pltpu.VMEM((C_VF, F), f32),
            pltpu.VMEM((C_VF, F), f32),
            pltpu.VMEM((C_VF, F), f32),
            pltpu.VMEM((C_VF, F), f32),
            pltpu.VMEM((C_L, K), f32),
            pltpu.VMEM((C_L, K), f32),
            pltpu.VMEM((C_L, K), f32),
            pltpu.VMEM((C_L, K), f32),
            pltpu.VMEM((C_L, K), f32),
            pltpu.VMEM((C_L, K), f32),
            pltpu.VMEM((C_L, K), f32),
            pltpu.VMEM((C_L, K), f32),
            pltpu.VMEM((C_L, K), f32),
            pltpu.VMEM((C_L, D), f32),
            pltpu.VMEM((C_L, D), f32),
            pltpu.VMEM((C_L, D), f32),
            pltpu.VMEM((C_L, 16), f32),
            pltpu.VMEM((C_L, 16), f32),
            pltpu.VMEM((C_L, 16), f32),
            pltpu.VMEM((RPW,), f32),
            pltpu.VMEM((RPW,), f32),
            pltpu.VMEM((RPW,), f32),
            pltpu.SemaphoreType.DMA,
            pltpu.SemaphoreType.DMA,
            pltpu.SemaphoreType.DMA,
            pltpu.SemaphoreType.DMA,
            pltpu.SemaphoreType.DMA,
            pltpu.SemaphoreType.DMA,
            pltpu.SemaphoreType.DMA,
            pltpu.SemaphoreType.DMA,
            pltpu.SemaphoreType.DMA,
            pltpu.SemaphoreType.DMA,
            pltpu.SemaphoreType.DMA,
        ],
    )(u, i, j, vf, ulat, ilat, uvisf, ibias)


BM = 1024


def _tc_body(dvf_ref, wt_ref, uvis_ref, latp_ref, bdiff_ref, out_ref):
    p = jnp.dot(dvf_ref[...].astype(jnp.bfloat16), wt_ref[...],
                preferred_element_type=jnp.float32)
    vis = jnp.sum(p[:, :D] * uvis_ref[...], axis=1)
    beta = jnp.sum(p[:, D:], axis=1)
    lat = jnp.sum(latp_ref[...], axis=1)
    out_ref[0, 0, :] = vis + beta + lat + bdiff_ref[0, 0, :]


def _tc_combine(dvf, wt, uvis, latp, bdiff2d):
    grid = (B // BM,)
    return pl.pallas_call(
        _tc_body,
        grid=grid,
        in_specs=[
            pl.BlockSpec((BM, F), lambda m: (m, 0)),
            pl.BlockSpec((F, DP), lambda m: (0, 0)),
            pl.BlockSpec((BM, D), lambda m: (m, 0)),
            pl.BlockSpec((BM, 16), lambda m: (m, 0)),
            pl.BlockSpec((1, 1, BM), lambda m: (m, 0, 0)),
        ],
        out_specs=pl.BlockSpec((1, 1, BM), lambda m: (m, 0, 0)),
        out_shape=jax.ShapeDtypeStruct((B // BM, 1, BM), jnp.float32),
    )(dvf, wt, uvis, latp, bdiff2d)


@jax.jit
def kernel(trg_batch, U_latent_factors, I_latent_factors, U_visual_factors,
           W_proj, b_proj, beta_dash, user_bias, item_bias, visual_features):
    u = trg_batch[:, 0].astype(jnp.int32)
    i = trg_batch[:, 1].astype(jnp.int32)
    j = trg_batch[:, 2].astype(jnp.int32)

    dvf, uvis, latp, bdiff = _sc_gather(
        u, i, j, visual_features, U_latent_factors, I_latent_factors,
        U_visual_factors, item_bias)

    wt = jnp.concatenate(
        [W_proj, beta_dash, jnp.zeros((DP - D - 1, F), jnp.float32)],
        axis=0).T.astype(jnp.bfloat16)

    out3d = _tc_combine(dvf, wt, uvis, latp,
                        bdiff.reshape(B // BM, 1, BM))
    return out3d.reshape(B)


# speedup vs baseline: 1.9314x; 1.0119x over previous
import functools
import jax
import jax.numpy as jnp
from jax import lax
from jax.experimental import pallas as pl
from jax.experimental.pallas import tpu as pltpu
from jax.experimental.pallas import tpu_sc as plsc

N_U = 100000
N_I = 100000
K = 128
D = 256
F = 1024
B = 16384

NW = 32
C_VF = 8
C_L = 32
DP = 384


def _make_sc_body(nb):
    rpw = nb // NW
    n_vf = rpw // C_VF
    n_l = rpw // C_L

    def _sc_body(u_hbm, i_hbm, j_hbm, vf_hbm, ulat_hbm, ilat_hbm, uvis_hbm,
                 ibias_hbm, dvf_out, uvis_out, latp_out, bdiff_out,
                 idx_u, idx_i, idx_j,
                 va0, va1, vb0, vb1, vo0, vo1,
                 ul0, ul1, ul2, li0, li1, li2, lj0, lj1, lj2,
                 uv0, uv1, uv2, lp0, lp1, lp2,
                 bias_i_v, bias_j_v, bdiff_v,
                 sidx, vg0, vg1, vw0, vw1, lg0, lg1, lg2, lw0, lw1, lw2):
        wid = lax.axis_index("s") * 2 + lax.axis_index("c")
        base = wid * rpw

        pltpu.sync_copy(u_hbm.at[pl.ds(base, rpw)], idx_u)
        pltpu.sync_copy(i_hbm.at[pl.ds(base, rpw)], idx_i)
        pltpu.sync_copy(j_hbm.at[pl.ds(base, rpw)], idx_j)

        va = (va0, va1)
        vb = (vb0, vb1)
        vo = (vo0, vo1)
        vg = (vg0, vg1)
        vw = (vw0, vw1)

        def vf_gather(c, slot):
            off = c * C_VF
            pltpu.async_copy(vf_hbm.at[idx_i.at[pl.ds(off, C_VF)]], va[slot],
                             vg[slot])
            pltpu.async_copy(vf_hbm.at[idx_j.at[pl.ds(off, C_VF)]], vb[slot],
                             vg[slot])

        def vf_gather_wait(c, slot):
            off = c * C_VF
            pltpu.make_async_copy(vf_hbm.at[idx_i.at[pl.ds(off, C_VF)]],
                                  va[slot], vg[slot]).wait()
            pltpu.make_async_copy(vf_hbm.at[idx_j.at[pl.ds(off, C_VF)]],
                                  vb[slot], vg[slot]).wait()

        def vf_write(c, slot):
            pltpu.async_copy(
                vo[slot], dvf_out.at[pl.ds(base + c * C_VF, C_VF)], vw[slot])

        def vf_write_wait(c, slot):
            pltpu.make_async_copy(
                vo[slot], dvf_out.at[pl.ds(base + c * C_VF, C_VF)],
                vw[slot]).wait()

        vf_gather(0, 0)
        vf_gather(1, 1)

        cpi = pltpu.async_copy(ibias_hbm.at[idx_i], bias_i_v, sidx)
        cpj = pltpu.async_copy(ibias_hbm.at[idx_j], bias_j_v, sidx)
        cpi.wait()
        cpj.wait()
        for k in range(rpw // 16):
            s = pl.ds(k * 16, 16)
            bdiff_v[s] = bias_i_v[s] - bias_j_v[s]
        pltpu.sync_copy(bdiff_v, bdiff_out.at[pl.ds(base, rpw)])

        def vf_visit(c, slot):
            @pl.when(c >= 2)
            def _():
                vf_write_wait(c - 2, slot)
            vf_gather_wait(c, slot)

            def sub_row(r, _):
                for k in range(F // 16):
                    s = pl.ds(k * 16, 16)
                    vo[slot][r, s] = va[slot][r, s] - vb[slot][r, s]
                return 0
            lax.fori_loop(0, C_VF, sub_row, 0)

            @pl.when(c + 2 < n_vf)
            def _():
                vf_gather(c + 2, slot)
            vf_write(c, slot)

        def vf_iter(c2, _):
            c = c2 * 2
            vf_visit(c, 0)
            vf_visit(c + 1, 1)
            return 0
        lax.fori_loop(0, n_vf // 2, vf_iter, 0)
        vf_write_wait(n_vf - 2, 0)
        vf_write_wait(n_vf - 1, 1)

        ul = (ul0, ul1, ul2)
        li = (li0, li1, li2)
        lj = (lj0, lj1, lj2)
        uv = (uv0, uv1, uv2)
        lp = (lp0, lp1, lp2)
        lg = (lg0, lg1, lg2)
        lw = (lw0, lw1, lw2)

        def l_gather(c, slot):
            off = c * C_L
            iu = idx_u.at[pl.ds(off, C_L)]
            ii = idx_i.at[pl.ds(off, C_L)]
            ij = idx_j.at[pl.ds(off, C_L)]
            pltpu.async_copy(ulat_hbm.at[iu], ul[slot], lg[slot])
            pltpu.async_copy(ilat_hbm.at[ii], li[slot], lg[slot])
            pltpu.async_copy(ilat_hbm.at[ij], lj[slot], lg[slot])
            pltpu.async_copy(uvis_hbm.at[iu], uv[slot], lg[slot])

        def l_gather_wait(c, slot):
            off = c * C_L
            iu = idx_u.at[pl.ds(off, C_L)]
            ii = idx_i.at[pl.ds(off, C_L)]
            ij = idx_j.at[pl.ds(off, C_L)]
            pltpu.make_async_copy(ulat_hbm.at[iu], ul[slot], lg[slot]).wait()
            pltpu.make_async_copy(ilat_hbm.at[ii], li[slot], lg[slot]).wait()
            pltpu.make_async_copy(ilat_hbm.at[ij], lj[slot], lg[slot]).wait()
            pltpu.make_async_copy(uvis_hbm.at[iu], uv[slot], lg[slot]).wait()

        def l_write(c, slot):
            off = base + c * C_L
            pltpu.async_copy(lp[slot], latp_out.at[pl.ds(off, C_L)], lw[slot])
            pltpu.async_copy(uv[slot], uvis_out.at[pl.ds(off, C_L)], lw[slot])

        def l_write_wait(c, slot):
            off = base + c * C_L
            pltpu.make_async_copy(lp[slot], latp_out.at[pl.ds(off, C_L)],
                                  lw[slot]).wait()
            pltpu.make_async_copy(uv[slot], uvis_out.at[pl.ds(off, C_L)],
                                  lw[slot]).wait()

        l_gather(0, 0)
        l_gather(1, 1)
        for c in range(n_l):
            slot = c % 3
            l_gather_wait(c, slot)

            def dot_row(r, _):
                acc = jnp.zeros((16,), jnp.float32)
                for k in range(K // 16):
                    s = pl.ds(k * 16, 16)
                    acc = acc + ul[slot][r, s] * (li[slot][r, s]
                                                  - lj[slot][r, s])
                lp[slot][r, pl.ds(0, 16)] = acc
                return 0
            lax.fori_loop(0, C_L, dot_row, 0)

            l_write(c, slot)
            if c + 2 < n_l:
                nslot = (c + 2) % 3
                if c - 1 >= 0:
                    l_write_wait(c - 1, nslot)
                l_gather(c + 2, nslot)
        for c in range(max(0, n_l - 3), n_l):
            l_write_wait(c, c % 3)

    return _sc_body


def _sc_gather(u, i, j, vf, ulat, ilat, uvisf, ibias):
    nb = u.shape[0]
    rpw = nb // NW
    mesh = plsc.VectorSubcoreMesh(core_axis_name="c", subcore_axis_name="s")
    f32 = jnp.float32
    return pl.kernel(
        _make_sc_body(nb),
        out_type=[
            jax.ShapeDtypeStruct((nb, F), f32),
            jax.ShapeDtypeStruct((nb, D), f32),
            jax.ShapeDtypeStruct((nb, 16), f32),
            jax.ShapeDtypeStruct((nb,), f32),
        ],
        mesh=mesh,
        scratch_types=[
            pltpu.VMEM((rpw,), jnp.int32),
            pltpu.VMEM((rpw,), jnp.int32),
            pltpu.VMEM((rpw,), jnp.int32),
            pltpu.VMEM((C_VF, F), f32),
            pltpu.VMEM((C_VF, F), f32),
            pltpu.VMEM((C_VF, F), f32),
            pltpu.VMEM((C_VF, F), f32),
            pltpu.VMEM((C_VF, F), f32),
            pltpu.VMEM((C_VF, F), f32),
            pltpu.VMEM((C_L, K), f32),
            pltpu.VMEM((C_L, K), f32),
            pltpu.VMEM((C_L, K), f32),
            pltpu.VMEM((C_L, K), f32),
            pltpu.VMEM((C_L, K), f32),
            pltpu.VMEM((C_L, K), f32),
            pltpu.VMEM((C_L, K), f32),
            pltpu.VMEM((C_L, K), f32),
            pltpu.VMEM((C_L, K), f32),
            pltpu.VMEM((C_L, D), f32),
            pltpu.VMEM((C_L, D), f32),
            pltpu.VMEM((C_L, D), f32),
            pltpu.VMEM((C_L, 16), f32),
            pltpu.VMEM((C_L, 16), f32),
            pltpu.VMEM((C_L, 16), f32),
            pltpu.VMEM((rpw,), f32),
            pltpu.VMEM((rpw,), f32),
            pltpu.VMEM((rpw,), f32),
            pltpu.SemaphoreType.DMA,
            pltpu.SemaphoreType.DMA,
            pltpu.SemaphoreType.DMA,
            pltpu.SemaphoreType.DMA,
            pltpu.SemaphoreType.DMA,
            pltpu.SemaphoreType.DMA,
            pltpu.SemaphoreType.DMA,
            pltpu.SemaphoreType.DMA,
            pltpu.SemaphoreType.DMA,
            pltpu.SemaphoreType.DMA,
            pltpu.SemaphoreType.DMA,
        ],
    )(u, i, j, vf, ulat, ilat, uvisf, ibias)


BM = 1024


def _tc_body(dvf_ref, wt_ref, uvis_ref, latp_ref, bdiff_ref, out_ref):
    p = jnp.dot(dvf_ref[...].astype(jnp.bfloat16), wt_ref[...],
                preferred_element_type=jnp.float32)
    vis = jnp.sum(p[:, :D] * uvis_ref[...], axis=1)
    beta = jnp.sum(p[:, D:], axis=1)
    lat = jnp.sum(latp_ref[...], axis=1)
    out_ref[0, 0, :] = vis + beta + lat + bdiff_ref[0, 0, :]


def _tc_combine(dvf, wt, uvis, latp, bdiff3d):
    nb = dvf.shape[0]
    grid = (nb // BM,)
    return pl.pallas_call(
        _tc_body,
        grid=grid,
        in_specs=[
            pl.BlockSpec((BM, F), lambda m: (m, 0)),
            pl.BlockSpec((F, DP), lambda m: (0, 0)),
            pl.BlockSpec((BM, D), lambda m: (m, 0)),
            pl.BlockSpec((BM, 16), lambda m: (m, 0)),
            pl.BlockSpec((1, 1, BM), lambda m: (m, 0, 0)),
        ],
        out_specs=pl.BlockSpec((1, 1, BM), lambda m: (m, 0, 0)),
        out_shape=jax.ShapeDtypeStruct((nb // BM, 1, BM), jnp.float32),
    )(dvf, wt, uvis, latp, bdiff3d)


N_SPLIT = 2


@jax.jit
def kernel(trg_batch, U_latent_factors, I_latent_factors, U_visual_factors,
           W_proj, b_proj, beta_dash, user_bias, item_bias, visual_features):
    u = trg_batch[:, 0].astype(jnp.int32)
    i = trg_batch[:, 1].astype(jnp.int32)
    j = trg_batch[:, 2].astype(jnp.int32)

    wt = jnp.concatenate(
        [W_proj, beta_dash, jnp.zeros((DP - D - 1, F), jnp.float32)],
        axis=0).T.astype(jnp.bfloat16)

    h = B // N_SPLIT
    outs = []
    for p in range(N_SPLIT):
        sl = slice(p * h, (p + 1) * h)
        dvf, uvis, latp, bdiff = _sc_gather(
            u[sl], i[sl], j[sl], visual_features, U_latent_factors,
            I_latent_factors, U_visual_factors, item_bias)
        outs.append(_tc_combine(dvf, wt, uvis, latp,
                                bdiff.reshape(h // BM, 1, BM)))
    return jnp.concatenate([o.reshape(h) for o in outs])
